# baseline (speedup 1.0000x reference)
import functools

import jax
import jax.numpy as jnp
from jax import lax
from jax.experimental import pallas as pl
from jax.experimental.pallas import tpu as pltpu

_N = 131072
_D = 512
_K = 1310
_KP = 1536
_GB = 1024


def _gate_body(x_ref, wg_ref, bg_ref, g_ref, out_ref):
    xb = x_ref[...]
    w = wg_ref[...]
    s = None
    for j in range(4):
        p = lax.dot_general(w[:, j * 128:(j + 1) * 128],
                            xb[:, j * 128:(j + 1) * 128],
                            (((1,), (1,)), ((), ())),
                            preferred_element_type=jnp.float32)
        s = p if s is None else s + p
    g_ref[...] = s + bg_ref[...][0]
    out_ref[...] = jnp.zeros_like(out_ref)


def _gate(x, wg_row, bg):
    return pl.pallas_call(
        _gate_body,
        grid=(_N // _GB,),
        in_specs=[
            pl.BlockSpec((_GB, _D), lambda i: (i, 0)),
            pl.BlockSpec((1, _D), lambda i: (0, 0)),
            pl.BlockSpec((1,), lambda i: (0,)),
        ],
        out_specs=[
            pl.BlockSpec((1, _GB), lambda i: (0, i)),
            pl.BlockSpec((_GB, _D), lambda i: (i, 0)),
        ],
        out_shape=[
            jax.ShapeDtypeStruct((1, _N), jnp.float32),
            jax.ShapeDtypeStruct((_N, _D), jnp.float32),
        ],
    )(x, wg_row, bg)


def _key_of(gf):
    b = lax.bitcast_convert_type(gf, jnp.int32)
    return b ^ (jnp.int32(0x7FFFFFFF) & (b >> 31))


def _thresh_body(g_ref, o_ref):
    keys = _key_of(g_ref[...])

    def step(i, t):
        b = 31 - i
        cand = t + (jnp.int32(1) << b)
        cnt = jnp.sum((keys >= cand).astype(jnp.int32))
        return jnp.where(cnt >= _K, cand, t)

    t = lax.fori_loop(0, 32, step, jnp.int32(-2147483648))
    cgt = jnp.sum((keys > t).astype(jnp.int32))
    need = _K - cgt
    rows = lax.broadcasted_iota(jnp.int32, (8, 128), 0)
    o_ref[...] = jnp.where(rows == 0, t,
                  jnp.where(rows == 1, cgt,
                   jnp.where(rows == 2, need, 0)))


def _threshold(g):
    return pl.pallas_call(
        _thresh_body,
        out_shape=jax.ShapeDtypeStruct((8, 128), jnp.int32),
    )(g.reshape(_N // 128, 128))


def _mlp_body(ax_ref, w1_ref, b1_ref, w2_ref, b2_ref, o_ref):
    h = jnp.maximum(
        jnp.dot(ax_ref[...], w1_ref[...],
                preferred_element_type=jnp.float32) + b1_ref[...], 0.0)
    o_ref[...] = jnp.dot(h, w2_ref[...],
                         preferred_element_type=jnp.float32) + b2_ref[...]


def _mlp(ax, w1, b1_row, w2, b2_row):
    return pl.pallas_call(
        _mlp_body,
        out_shape=jax.ShapeDtypeStruct((_KP, _D), jnp.float32),
    )(ax, w1, b1_row, w2, b2_row)


def kernel(x, Wg, bg, W1, b1, W2, b2):
    g2, out0 = _gate(x, Wg.reshape(1, _D), bg)
    g = g2.reshape(_N)
    thr = _threshold(g)
    del thr
    _, idx = lax.top_k(g, _K)
    ax = jnp.zeros((_KP, _D), jnp.float32).at[: _K].set(x[idx])
    outa = _mlp(ax, W1, b1.reshape(1, _D), W2, b2.reshape(1, _D))
    output = out0.at[idx].set(outa[: _K])
    mask = jnp.zeros((_N,), jnp.float32).at[idx].set(1.0)
    return output, mask


# speedup vs baseline: 1.9423x; 1.9423x over previous
import functools

import jax
import jax.numpy as jnp
from jax import lax
from jax.experimental import pallas as pl
from jax.experimental.pallas import tpu as pltpu
from jax.experimental.pallas import tpu_sc as plsc

_N = 131072
_D = 512
_K = 1310
_KP = 1536
_GB = 1024

_NT = 16
_CHUNK = _N // _NT
_ROWS = _CHUNK // 128
_IDXBUF = 2048
_DUMP = 2047

_NW = 32
_GROWS = _KP // _NW

_SC_PARAMS = pltpu.CompilerParams(needs_layout_passes=False)


def _gate_body(x_ref, wg_ref, bg_ref, g_ref, out_ref):
    xb = x_ref[...]
    w = wg_ref[...]
    s = None
    for j in range(4):
        p = lax.dot_general(w[:, j * 128:(j + 1) * 128],
                            xb[:, j * 128:(j + 1) * 128],
                            (((1,), (1,)), ((), ())),
                            preferred_element_type=jnp.float32)
        s = p if s is None else s + p
    g_ref[...] = s + bg_ref[...][0]
    out_ref[...] = jnp.zeros_like(out_ref)


def _gate(x, wg_row, bg):
    return pl.pallas_call(
        _gate_body,
        grid=(_N // _GB,),
        in_specs=[
            pl.BlockSpec((_GB, _D), lambda i: (i, 0)),
            pl.BlockSpec((1, _D), lambda i: (0, 0)),
            pl.BlockSpec((1,), lambda i: (0,)),
        ],
        out_specs=[
            pl.BlockSpec((1, _GB), lambda i: (0, i)),
            pl.BlockSpec((_GB, _D), lambda i: (i, 0)),
        ],
        out_shape=[
            jax.ShapeDtypeStruct((1, _N), jnp.float32),
            jax.ShapeDtypeStruct((_N, _D), jnp.float32),
        ],
    )(x, wg_row, bg)


def _key_of(gf):
    b = lax.bitcast_convert_type(gf, jnp.int32)
    return b ^ (jnp.int32(0x7FFFFFFF) & (b >> 31))


def _thresh_body(g_ref, o_ref):
    keys = _key_of(g_ref[...])

    def step(i, t):
        b = 31 - i
        cand = t + (jnp.int32(1) << b)
        cnt = jnp.sum((keys >= cand).astype(jnp.int32))
        return jnp.where(cnt >= _K, cand, t)

    t = lax.fori_loop(0, 32, step, jnp.int32(-2147483648))
    cgt = jnp.sum((keys > t).astype(jnp.int32))
    need = _K - cgt
    rows = lax.broadcasted_iota(jnp.int32, (8, 128), 0)
    o_ref[...] = jnp.where(rows == 0, t,
                  jnp.where(rows == 1, cgt,
                   jnp.where(rows == 2, need, 0)))


def _threshold(g):
    return pl.pallas_call(
        _thresh_body,
        out_shape=jax.ShapeDtypeStruct((8, 128), jnp.int32),
    )(g.reshape(_N // 128, 128))


def _compact_body(g_hbm, thr_hbm, idx_hbm, mask_hbm,
                  gv, maskv, tv, needv, myc, cntv, privbuf,
                  cv, fv, padv, cnt_spm, mrg_spm, pad_spm):
    wid = lax.axis_index("s")
    base_el = wid * _CHUNK
    lanes = lax.iota(jnp.int32, 16)
    zero16 = jnp.zeros((16,), jnp.int32)

    pltpu.sync_copy(g_hbm.at[pl.ds(base_el, _CHUNK)], gv)
    pltpu.sync_copy(thr_hbm.at[pl.ds(0, 16)], tv)
    pltpu.sync_copy(thr_hbm.at[pl.ds(256, 16)], needv)
    tvec = tv[...]
    needvec = needv[...]

    def zrow(i, _):
        privbuf[pl.ds(pl.multiple_of(i * 16, 16), 16)] = zero16
        return 0

    lax.fori_loop(0, _IDXBUF // 16, zrow, 0)

    def p1(j, carry):
        cgt, ceq = carry
        off = pl.multiple_of(j * 16, 16)
        key = _key_of(gv[pl.ds(off, 16)])
        cgt = cgt + plsc.all_reduce_population_count(key > tvec)
        ceq = ceq + plsc.all_reduce_population_count(key == tvec)
        return cgt, ceq

    cgt, ceq = lax.fori_loop(0, _CHUNK // 16, p1, (zero16, zero16))
    myc[...] = cgt
    pltpu.sync_copy(myc, cnt_spm.at[0, wid])
    myc[...] = ceq
    pltpu.sync_copy(myc, cnt_spm.at[1, wid])
    plsc.subcore_barrier()

    pltpu.sync_copy(cnt_spm, cntv)
    base_gt = zero16
    base_eq = zero16
    total_gt = zero16
    for w in range(_NT):
        rg = cntv[0, w]
        re = cntv[1, w]
        mine = jnp.int32(w) < wid
        base_gt = base_gt + jnp.where(mine, rg, zero16)
        base_eq = base_eq + jnp.where(mine, re, zero16)
        total_gt = total_gt + rg

    def p2(j, carry):
        rgt, req = carry
        off = pl.multiple_of(j * 16, 16)
        key = _key_of(gv[pl.ds(off, 16)])
        mgt = key > tvec
        meq = key == tvec
        igt = mgt.astype(jnp.int32)
        ieq = meq.astype(jnp.int32)
        cs_gt = plsc.cumsum(igt)
        cs_eq = plsc.cumsum(ieq)
        grank_eq = req + (cs_eq - ieq)
        sel_eq = meq & (grank_eq < needvec)
        sel = mgt | sel_eq
        pos = jnp.where(mgt, rgt + (cs_gt - igt), total_gt + grank_eq)
        pos = jnp.minimum(jnp.maximum(pos, 0), jnp.int32(_IDXBUF - 1))
        plsc.store_scatter(privbuf, [pos], base_el + off + lanes, mask=sel)
        maskv[pl.ds(off, 16)] = sel.astype(jnp.float32)
        rgt = rgt + plsc.all_reduce_population_count(mgt)
        req = req + plsc.all_reduce_population_count(meq)
        return rgt, req

    lax.fori_loop(0, _CHUNK // 16, p2, (base_gt, base_eq))
    pltpu.sync_copy(maskv, mask_hbm.at[pl.ds(base_el, _CHUNK)])

    pltpu.sync_copy(privbuf, mrg_spm.at[wid])
    plsc.subcore_barrier()

    nmerge = _KP // 128
    pltpu.sync_copy(mrg_spm.at[:, pl.ds(wid * 128, 128)], cv)
    for c in range(8):
        acc = zero16
        for w in range(_NT):
            acc = acc + cv[w, pl.ds(c * 16, 16)]
        fv[pl.ds(c * 16, 16)] = acc

    @pl.when(wid == 0)
    def _():
        pltpu.sync_copy(fv.at[pl.ds(0, 16)], pad_spm)
    plsc.subcore_barrier()

    @pl.when(wid >= _K // 128)
    def _():
        pltpu.sync_copy(pad_spm, padv)
        pv = padv[...]
        for c in range(8):
            posvec = wid * 128 + c * 16 + lanes
            cur = fv[pl.ds(c * 16, 16)]
            fv[pl.ds(c * 16, 16)] = jnp.where(posvec < _K, cur, pv)

    @pl.when(wid < nmerge)
    def _():
        pltpu.sync_copy(fv, idx_hbm.at[pl.ds(wid * 128, 128)])


_compact = pl.kernel(
    _compact_body,
    out_type=[
        jax.ShapeDtypeStruct((_KP,), jnp.int32),
        jax.ShapeDtypeStruct((_N,), jnp.float32),
    ],
    mesh=plsc.VectorSubcoreMesh(
        core_axis_name="c", subcore_axis_name="s", num_cores=1),
    compiler_params=_SC_PARAMS,
    scratch_types=[
        pltpu.VMEM((_CHUNK,), jnp.float32),
        pltpu.VMEM((_CHUNK,), jnp.float32),
        pltpu.VMEM((16,), jnp.int32),
        pltpu.VMEM((16,), jnp.int32),
        pltpu.VMEM((16,), jnp.int32),
        pltpu.VMEM((2, _NT, 16), jnp.int32),
        pltpu.VMEM((_IDXBUF,), jnp.int32),
        pltpu.VMEM((_NT, 128), jnp.int32),
        pltpu.VMEM((128,), jnp.int32),
        pltpu.VMEM((16,), jnp.int32),
        pltpu.VMEM_SHARED((2, _NT, 16), jnp.int32),
        pltpu.VMEM_SHARED((_NT, _IDXBUF), jnp.int32),
        pltpu.VMEM_SHARED((16,), jnp.int32),
    ],
)


def _gather_body(x_hbm, idx_hbm, ax_hbm, idx_v, rows_v, sem):
    wid = lax.axis_index("s") * 2 + lax.axis_index("c")
    base = wid * _GROWS
    pltpu.sync_copy(idx_hbm.at[pl.ds(base, _GROWS)], idx_v)
    pltpu.async_copy(x_hbm.at[idx_v], rows_v, sem).wait()
    pltpu.sync_copy(rows_v, ax_hbm.at[pl.ds(base, _GROWS)])


_gather = pl.kernel(
    _gather_body,
    out_type=jax.ShapeDtypeStruct((_KP, _D), jnp.float32),
    mesh=plsc.VectorSubcoreMesh(core_axis_name="c", subcore_axis_name="s"),
    compiler_params=_SC_PARAMS,
    scratch_types=[
        pltpu.VMEM((_GROWS,), jnp.int32),
        pltpu.VMEM((_GROWS, _D), jnp.float32),
        pltpu.SemaphoreType.DMA,
    ],
)


def _mlp_body(ax_ref, w1_ref, b1_ref, w2_ref, b2_ref, o_ref):
    h = jnp.maximum(
        jnp.dot(ax_ref[...], w1_ref[...],
                preferred_element_type=jnp.float32) + b1_ref[...], 0.0)
    o_ref[...] = jnp.dot(h, w2_ref[...],
                         preferred_element_type=jnp.float32) + b2_ref[...]


def _mlp(ax, w1, b1_row, w2, b2_row):
    return pl.pallas_call(
        _mlp_body,
        out_shape=jax.ShapeDtypeStruct((_KP, _D), jnp.float32),
    )(ax, w1, b1_row, w2, b2_row)


def _scatter_body(outa_hbm, idx_hbm, out_hbm, idx_v, rows_v, sem):
    wid = lax.axis_index("s") * 2 + lax.axis_index("c")
    base = wid * _GROWS
    pltpu.sync_copy(idx_hbm.at[pl.ds(base, _GROWS)], idx_v)
    pltpu.sync_copy(outa_hbm.at[pl.ds(base, _GROWS)], rows_v)
    pltpu.async_copy(rows_v, out_hbm.at[idx_v], sem).wait()


_scatter = pl.kernel(
    _scatter_body,
    out_type=(),
    mesh=plsc.VectorSubcoreMesh(core_axis_name="c", subcore_axis_name="s"),
    compiler_params=_SC_PARAMS,
    scratch_types=[
        pltpu.VMEM((_GROWS,), jnp.int32),
        pltpu.VMEM((_GROWS, _D), jnp.float32),
        pltpu.SemaphoreType.DMA,
    ],
)


def kernel(x, Wg, bg, W1, b1, W2, b2):
    g2, out0 = _gate(x, Wg.reshape(1, _D), bg)
    g = g2.reshape(_N)
    thr = _threshold(g).reshape(1024)
    idx, mask = _compact(g, thr)
    ax = _gather(x, idx)
    outa = _mlp(ax, W1, b1.reshape(1, _D), W2, b2.reshape(1, _D))
    oref = jax.new_ref(out0)
    _scatter(outa, idx, oref)
    return jax.freeze(oref), mask
